# Initial kernel scaffold; baseline (speedup 1.0000x reference)
import jax
import jax.numpy as jnp
from jax.experimental import pallas as pl


def kernel(tree_x, tree_edge_index, graph_x, graph_edge_index, params):
    raise NotImplementedError("write your pallas kernel here")


# speedup vs baseline: 4.0695x; 4.0695x over previous
import functools

import jax
import jax.numpy as jnp
from jax import lax
from jax.experimental import pallas as pl
from jax.experimental.pallas import tpu as pltpu
from jax.experimental.pallas import tpu_sc as plsc

D = 128
NC, NS = 2, 16
NW = NC * NS
CHUNK = 128
BLK = 2048

ZDIM = 56
VOCAB = 8192
MAX_TREE_NODES = 12
N_LAYERS = 3

_TREE = dict(n=5000, n_pad=6144, n_chunks=3)
_GRAPH = dict(n=10000, n_pad=10240, n_chunks=79)


def _make_edge_pass(n_pad, n_chunks):
    rpt = n_pad // NS
    assert rpt % CHUNK == 0
    mesh = plsc.VectorSubcoreMesh(core_axis_name="c", subcore_axis_name="s",
                                  num_cores=NC, num_subcores=NS)

    def body(p_hbm, q_hbm, ei_hbm, out_hbm,
             idx_v, buf_p, buf_q, aggr, sem_p, sem_q):
        cid = lax.axis_index("c")
        sid = lax.axis_index("s")
        wid = cid * NS + sid

        zeros = jnp.zeros((16,), jnp.float32)

        @pl.loop(0, CHUNK)
        def _(r):
            for k in range(D // 16):
                buf_p[r, pl.ds(k * 16, 16)] = zeros

        @pl.loop(0, rpt // CHUNK)
        def _(b):
            pltpu.sync_copy(buf_p, aggr.at[pl.ds(sid * rpt + b * CHUNK, CHUNK)])

        plsc.subcore_barrier()

        @pl.loop(0, n_chunks)
        def _(c):
            pltpu.sync_copy(ei_hbm.at[wid, c], idx_v)
            cp_p = pltpu.async_copy(p_hbm.at[idx_v.at[1]], buf_p, sem_p)
            cp_q = pltpu.async_copy(q_hbm.at[idx_v.at[0]], buf_q, sem_q)
            cp_p.wait()
            cp_q.wait()

            @pl.loop(0, CHUNK)
            def _(r):
                for k in range(D // 16):
                    sl = pl.ds(k * 16, 16)
                    buf_q[r, sl] = jnp.maximum(buf_p[r, sl] + buf_q[r, sl], 0.0)

            pltpu.sync_copy(buf_q, aggr.at[idx_v.at[1]], add=True)

        plsc.subcore_barrier()
        pltpu.sync_copy(aggr.at[pl.ds(sid * rpt, rpt)],
                        out_hbm.at[cid, pl.ds(sid * rpt, rpt)])

    return pl.kernel(
        body,
        out_type=jax.ShapeDtypeStruct((NC, n_pad, D), jnp.float32),
        mesh=mesh,
        scratch_types=[
            pltpu.VMEM((2, CHUNK), jnp.int32),
            pltpu.VMEM((CHUNK, D), jnp.float32),
            pltpu.VMEM((CHUNK, D), jnp.float32),
            pltpu.VMEM_SHARED((n_pad, D), jnp.float32),
            pltpu.SemaphoreType.DMA,
            pltpu.SemaphoreType.DMA,
        ],
    )


def _dot(a, b):
    return jnp.dot(a, b, preferred_element_type=jnp.float32)


def _proj_body(x_ref, wp_ref, bp_ref, wi_ref, bi_ref, wj_ref, p_ref, q_ref):
    h = jnp.maximum(_dot(x_ref[...], wp_ref[...]) + bp_ref[...], 0.0)
    p_ref[...] = _dot(h, wi_ref[...]) + bi_ref[...]
    q_ref[...] = _dot(h, wj_ref[...])


def _proj_call(n_pad, x, wp, bp, wi, bi, wj):
    g = n_pad // BLK
    row = pl.BlockSpec((BLK, D), lambda i: (i, 0))
    full = lambda s: pl.BlockSpec(s, lambda i: tuple(0 for _ in s))
    return pl.pallas_call(
        _proj_body,
        grid=(g,),
        in_specs=[row, full((D, D)), full((1, D)), full((D, D)), full((1, D)),
                  full((D, D))],
        out_specs=[row, row],
        out_shape=[jax.ShapeDtypeStruct((n_pad, D), jnp.float32)] * 2,
    )(x, wp, bp, wi, bi, wj)


def _mid_body(a_ref, wl_ref, bl_ref, wi_ref, bi_ref, wj_ref, p_ref, q_ref):
    a = a_ref[0] + a_ref[1]
    h = jnp.maximum(_dot(a, wl_ref[...]) + bl_ref[...], 0.0)
    p_ref[...] = _dot(h, wi_ref[...]) + bi_ref[...]
    q_ref[...] = _dot(h, wj_ref[...])


def _mid_call(n_pad, parts, wl, bl, wi, bi, wj):
    g = n_pad // BLK
    row = pl.BlockSpec((BLK, D), lambda i: (i, 0))
    part = pl.BlockSpec((NC, BLK, D), lambda i: (0, i, 0))
    full = lambda s: pl.BlockSpec(s, lambda i: tuple(0 for _ in s))
    return pl.pallas_call(
        _mid_body,
        grid=(g,),
        in_specs=[part, full((D, D)), full((1, D)), full((D, D)), full((1, D)),
                  full((D, D))],
        out_specs=[row, row],
        out_shape=[jax.ShapeDtypeStruct((n_pad, D), jnp.float32)] * 2,
    )(parts, wl, bl, wi, bi, wj)


def _pool_body(n_valid, a_ref, wl_ref, bl_ref, o_ref):
    i = pl.program_id(0)
    a = a_ref[0] + a_ref[1]
    h = jnp.maximum(_dot(a, wl_ref[...]) + bl_ref[...], 0.0)
    rows = lax.broadcasted_iota(jnp.int32, (BLK, 1), 0) + i * BLK
    h = jnp.where(rows < n_valid, h, 0.0)
    s = jnp.sum(h, axis=0, keepdims=True)

    @pl.when(i == 0)
    def _():
        o_ref[...] = s

    @pl.when(i > 0)
    def _():
        o_ref[...] += s


def _pool_call(n_pad, n_valid, parts, wl, bl):
    g = n_pad // BLK
    part = pl.BlockSpec((NC, BLK, D), lambda i: (0, i, 0))
    full = lambda s: pl.BlockSpec(s, lambda i: tuple(0 for _ in s))
    return pl.pallas_call(
        functools.partial(_pool_body, n_valid),
        grid=(g,),
        in_specs=[part, full((D, D)), full((1, D))],
        out_specs=pl.BlockSpec((1, D), lambda i: (0, 0)),
        out_shape=jax.ShapeDtypeStruct((1, D), jnp.float32),
    )(parts, wl, bl)


def _head_body(tsum, gsum, wmu, bmu, wlv, blv, eps, wz, bz, w1, b1, w2, b2,
               wnd, bnd, logits_ref, nf_ref, mu_ref, lv_ref):
    vec = jnp.concatenate(
        [tsum[...] * (1.0 / _TREE["n"]), gsum[...] * (1.0 / _GRAPH["n"])], axis=1)
    mu = _dot(vec, wmu[...]) + bmu[...]
    lv = _dot(vec, wlv[...]) + blv[...]
    std = jnp.exp(0.5 * lv)
    z = mu + eps[...] * std
    h = jnp.maximum(_dot(z, wz[...]) + bz[...], 0.0)
    f1 = jnp.maximum(_dot(h, w1[...]) + b1[...], 0.0)
    logits_ref[...] = _dot(f1, w2[...]) + b2[...]
    nf_ref[...] = _dot(h, wnd[...]) + bnd[...]
    mu_ref[...] = mu
    lv_ref[...] = lv


def _head_call(tsum, gsum, eps, p):
    args = (tsum, gsum,
            p["fc_mu_W"], p["fc_mu_b"].reshape(1, -1),
            p["fc_logvar_W"], p["fc_logvar_b"].reshape(1, -1),
            eps,
            p["z_to_hidden_W"], p["z_to_hidden_b"].reshape(1, -1),
            p["fp1_W"], p["fp1_b"].reshape(1, -1),
            p["fp2_W"], p["fp2_b"].reshape(1, -1),
            p["nd_W"], p["nd_b"].reshape(1, -1))
    return pl.pallas_call(
        _head_body,
        out_shape=[
            jax.ShapeDtypeStruct((1, VOCAB), jnp.float32),
            jax.ShapeDtypeStruct((1, 32), jnp.float32),
            jax.ShapeDtypeStruct((1, ZDIM), jnp.float32),
            jax.ShapeDtypeStruct((1, ZDIM), jnp.float32),
        ],
    )(*args)


def _prep_edges(ei, n, n_chunks):
    e_pad = NW * n_chunks * CHUNK
    pad = e_pad - ei.shape[1]
    src = jnp.concatenate([ei[0], jnp.zeros((pad,), jnp.int32)])
    dst = jnp.concatenate([ei[1], jnp.full((pad,), n, jnp.int32)])
    return jnp.concatenate(
        [src.reshape(NW, n_chunks, 1, CHUNK), dst.reshape(NW, n_chunks, 1, CHUNK)],
        axis=2)


def _encode(x, ei, p, pre, cfg):
    n, n_pad, n_chunks = cfg["n"], cfg["n_pad"], cfg["n_chunks"]
    ei4 = _prep_edges(ei, n, n_chunks)
    x_pad = jnp.pad(x, ((0, n_pad - n), (0, 0)))
    edge_pass = _make_edge_pass(n_pad, n_chunks)

    def msg_w(l):
        w = p[f"{pre}_l{l}_msg_W"]
        return w[:D], p[f"{pre}_l{l}_msg_b"].reshape(1, -1), w[D:]

    wi, bi, wj = msg_w(0)
    P, Q = _proj_call(n_pad, x_pad, p[pre + "_proj_W"],
                      p[pre + "_proj_b"].reshape(1, -1), wi, bi, wj)
    for l in range(N_LAYERS):
        parts = edge_pass(P, Q, ei4)
        wl = p[f"{pre}_l{l}_lin_W"]
        bl = p[f"{pre}_l{l}_lin_b"].reshape(1, -1)
        if l < N_LAYERS - 1:
            wi, bi, wj = msg_w(l + 1)
            P, Q = _mid_call(n_pad, parts, wl, bl, wi, bi, wj)
        else:
            return _pool_call(n_pad, n, parts, wl, bl)


def kernel(tree_x, tree_edge_index, graph_x, graph_edge_index, params):
    p = params
    tsum = _encode(tree_x, tree_edge_index, p, "tree", _TREE)
    gsum = _encode(graph_x, graph_edge_index, p, "graph", _GRAPH)
    eps = jax.random.normal(jax.random.key(42), (1, ZDIM), dtype=jnp.float32)
    logits, node_feats, mu, logvar = _head_call(tsum, gsum, eps, p)
    frags_logits = jnp.broadcast_to(logits[:, None, :], (1, MAX_TREE_NODES, VOCAB))
    return (frags_logits, node_feats, mu, logvar)


# speedup vs baseline: 6.4076x; 1.5746x over previous
import functools

import jax
import jax.numpy as jnp
from jax import lax
from jax.experimental import pallas as pl
from jax.experimental.pallas import tpu as pltpu
from jax.experimental.pallas import tpu_sc as plsc

D = 128
HD = D // 2
NC, NS = 2, 16
NW = NC * NS
CHUNK = 128
BLK = 1024

ZDIM = 56
VOCAB = 8192
MAX_TREE_NODES = 12
N_LAYERS = 3

_TREE = dict(n=5000, n_pad=5120, n_chunks=6)
_GRAPH = dict(n=10000, n_pad=10240, n_chunks=158)


def _make_edge_pass(n_pad, n_chunks):
    rpt = n_pad // NS
    assert rpt % 64 == 0
    mesh = plsc.VectorSubcoreMesh(core_axis_name="c", subcore_axis_name="s",
                                  num_cores=NC, num_subcores=NS)

    assert n_chunks % 2 == 0
    n_pairs = n_chunks // 2

    def body(p_hbm, q_hbm, ei_hbm, out_hbm,
             idx_v, bp0, bq0, bp1, bq1, aggr, sp0, sq0, sp1, sq1):
        cid = lax.axis_index("c")
        sid = lax.axis_index("s")
        slots = ((bp0, bq0, sp0, sq0), (bp1, bq1, sp1, sq1))

        zeros = jnp.zeros((16,), jnp.float32)

        @pl.loop(0, 64)
        def _(r):
            for k in range(HD // 16):
                bp0[r, pl.ds(k * 16, 16)] = zeros

        @pl.loop(0, rpt // 64)
        def _(b):
            pltpu.sync_copy(bp0.at[pl.ds(0, 64)],
                            aggr.at[pl.ds(sid * rpt + b * 64, 64)])

        plsc.subcore_barrier()

        pltpu.sync_copy(ei_hbm.at[sid], idx_v)

        def start(c, slot):
            bp, bq, sp, sq = slots[slot]
            pltpu.async_copy(p_hbm.at[cid].at[idx_v.at[c, 1]], bp, sp)
            pltpu.async_copy(q_hbm.at[cid].at[idx_v.at[c, 0]], bq, sq)

        def consume(c, slot):
            bp, bq, sp, sq = slots[slot]
            pltpu.make_async_copy(p_hbm.at[cid].at[idx_v.at[c, 1]], bp, sp).wait()
            pltpu.make_async_copy(q_hbm.at[cid].at[idx_v.at[c, 0]], bq, sq).wait()

            @pl.loop(0, CHUNK)
            def _(r):
                for k in range(HD // 16):
                    sl = pl.ds(k * 16, 16)
                    bq[r, sl] = jnp.maximum(bp[r, sl] + bq[r, sl], 0.0)

            pltpu.sync_copy(bq, aggr.at[idx_v.at[c, 1]], add=True)

        start(0, 0)

        @pl.loop(0, n_pairs)
        def _(t):
            c0 = 2 * t
            start(c0 + 1, 1)
            consume(c0, 0)

            @pl.when(t < n_pairs - 1)
            def _():
                start(c0 + 2, 0)

            consume(c0 + 1, 1)

        plsc.subcore_barrier()
        pltpu.sync_copy(aggr.at[pl.ds(sid * rpt, rpt)],
                        out_hbm.at[cid, pl.ds(sid * rpt, rpt)])

    return pl.kernel(
        body,
        out_type=jax.ShapeDtypeStruct((NC, n_pad, HD), jnp.float32),
        mesh=mesh,
        compiler_params=pltpu.CompilerParams(use_tc_tiling_on_sc=False),
        scratch_types=[
            pltpu.VMEM((n_chunks, 2, CHUNK), jnp.int32),
            pltpu.VMEM((CHUNK, HD), jnp.float32),
            pltpu.VMEM((CHUNK, HD), jnp.float32),
            pltpu.VMEM((CHUNK, HD), jnp.float32),
            pltpu.VMEM((CHUNK, HD), jnp.float32),
            pltpu.VMEM_SHARED((n_pad, HD), jnp.float32),
            pltpu.SemaphoreType.DMA,
            pltpu.SemaphoreType.DMA,
            pltpu.SemaphoreType.DMA,
            pltpu.SemaphoreType.DMA,
        ],
    )


def _dot(a, b):
    return jnp.dot(a, b, preferred_element_type=jnp.float32)


def _emit_pq(h, wi_ref, bi_ref, wj_ref, p_ref, q_ref):
    p_ref[0] = _dot(h, wi_ref[..., :HD]) + bi_ref[..., :HD]
    p_ref[1] = _dot(h, wi_ref[..., HD:]) + bi_ref[..., HD:]
    q_ref[0] = _dot(h, wj_ref[..., :HD])
    q_ref[1] = _dot(h, wj_ref[..., HD:])


def _proj_body(x_ref, wp_ref, bp_ref, wi_ref, bi_ref, wj_ref, p_ref, q_ref):
    h = jnp.maximum(_dot(x_ref[...], wp_ref[...]) + bp_ref[...], 0.0)
    _emit_pq(h, wi_ref, bi_ref, wj_ref, p_ref, q_ref)


def _proj_call(n_pad, x, wp, bp, wi, bi, wj):
    g = n_pad // BLK
    row = pl.BlockSpec((BLK, D), lambda i: (i, 0))
    half = pl.BlockSpec((NC, BLK, HD), lambda i: (0, i, 0))
    full = lambda s: pl.BlockSpec(s, lambda i: tuple(0 for _ in s))
    return pl.pallas_call(
        _proj_body,
        grid=(g,),
        in_specs=[row, full((D, D)), full((1, D)), full((D, D)), full((1, D)),
                  full((D, D))],
        out_specs=[half, half],
        out_shape=[jax.ShapeDtypeStruct((NC, n_pad, HD), jnp.float32)] * 2,
    )(x, wp, bp, wi, bi, wj)


def _mid_body(a_ref, wl_ref, bl_ref, wi_ref, bi_ref, wj_ref, p_ref, q_ref):
    a = jnp.concatenate([a_ref[0], a_ref[1]], axis=-1)
    h = jnp.maximum(_dot(a, wl_ref[...]) + bl_ref[...], 0.0)
    _emit_pq(h, wi_ref, bi_ref, wj_ref, p_ref, q_ref)


def _mid_call(n_pad, parts, wl, bl, wi, bi, wj):
    g = n_pad // BLK
    half = pl.BlockSpec((NC, BLK, HD), lambda i: (0, i, 0))
    full = lambda s: pl.BlockSpec(s, lambda i: tuple(0 for _ in s))
    return pl.pallas_call(
        _mid_body,
        grid=(g,),
        in_specs=[half, full((D, D)), full((1, D)), full((D, D)), full((1, D)),
                  full((D, D))],
        out_specs=[half, half],
        out_shape=[jax.ShapeDtypeStruct((NC, n_pad, HD), jnp.float32)] * 2,
    )(parts, wl, bl, wi, bi, wj)


def _pool_body(n_valid, a_ref, wl_ref, bl_ref, o_ref):
    i = pl.program_id(0)
    a = jnp.concatenate([a_ref[0], a_ref[1]], axis=-1)
    h = jnp.maximum(_dot(a, wl_ref[...]) + bl_ref[...], 0.0)
    rows = lax.broadcasted_iota(jnp.int32, (BLK, 1), 0) + i * BLK
    h = jnp.where(rows < n_valid, h, 0.0)
    s = jnp.sum(h, axis=0, keepdims=True)

    @pl.when(i == 0)
    def _():
        o_ref[...] = s

    @pl.when(i > 0)
    def _():
        o_ref[...] += s


def _pool_call(n_pad, n_valid, parts, wl, bl):
    g = n_pad // BLK
    half = pl.BlockSpec((NC, BLK, HD), lambda i: (0, i, 0))
    full = lambda s: pl.BlockSpec(s, lambda i: tuple(0 for _ in s))
    return pl.pallas_call(
        functools.partial(_pool_body, n_valid),
        grid=(g,),
        in_specs=[half, full((D, D)), full((1, D))],
        out_specs=pl.BlockSpec((1, D), lambda i: (0, 0)),
        out_shape=jax.ShapeDtypeStruct((1, D), jnp.float32),
    )(parts, wl, bl)


def _head_body(tsum, gsum, wmu, bmu, wlv, blv, eps, wz, bz, w1, b1, w2, b2,
               wnd, bnd, logits_ref, nf_ref, mu_ref, lv_ref):
    vec = jnp.concatenate(
        [tsum[...] * (1.0 / _TREE["n"]), gsum[...] * (1.0 / _GRAPH["n"])], axis=1)
    mu = _dot(vec, wmu[...]) + bmu[...]
    lv = _dot(vec, wlv[...]) + blv[...]
    std = jnp.exp(0.5 * lv)
    z = mu + eps[...] * std
    h = jnp.maximum(_dot(z, wz[...]) + bz[...], 0.0)
    f1 = jnp.maximum(_dot(h, w1[...]) + b1[...], 0.0)
    logits_ref[...] = _dot(f1, w2[...]) + b2[...]
    nf_ref[...] = _dot(h, wnd[...]) + bnd[...]
    mu_ref[...] = mu
    lv_ref[...] = lv


def _head_call(tsum, gsum, eps, p):
    args = (tsum, gsum,
            p["fc_mu_W"], p["fc_mu_b"].reshape(1, -1),
            p["fc_logvar_W"], p["fc_logvar_b"].reshape(1, -1),
            eps,
            p["z_to_hidden_W"], p["z_to_hidden_b"].reshape(1, -1),
            p["fp1_W"], p["fp1_b"].reshape(1, -1),
            p["fp2_W"], p["fp2_b"].reshape(1, -1),
            p["nd_W"], p["nd_b"].reshape(1, -1))
    return pl.pallas_call(
        _head_body,
        out_shape=[
            jax.ShapeDtypeStruct((1, VOCAB), jnp.float32),
            jax.ShapeDtypeStruct((1, 32), jnp.float32),
            jax.ShapeDtypeStruct((1, ZDIM), jnp.float32),
            jax.ShapeDtypeStruct((1, ZDIM), jnp.float32),
        ],
    )(*args)


def _prep_edges(ei, n, n_chunks):
    e_pad = NS * n_chunks * CHUNK
    pad = e_pad - ei.shape[1]
    src = jnp.concatenate([ei[0], jnp.zeros((pad,), jnp.int32)])
    dst = jnp.concatenate([ei[1], jnp.full((pad,), n, jnp.int32)])
    return jnp.concatenate(
        [src.reshape(NS, n_chunks, 1, CHUNK), dst.reshape(NS, n_chunks, 1, CHUNK)],
        axis=2)


def _encode(x, ei, p, pre, cfg):
    n, n_pad, n_chunks = cfg["n"], cfg["n_pad"], cfg["n_chunks"]
    ei4 = _prep_edges(ei, n, n_chunks)
    x_pad = jnp.pad(x, ((0, n_pad - n), (0, 0)))
    edge_pass = _make_edge_pass(n_pad, n_chunks)

    def msg_w(l):
        w = p[f"{pre}_l{l}_msg_W"]
        return w[:D], p[f"{pre}_l{l}_msg_b"].reshape(1, -1), w[D:]

    wi, bi, wj = msg_w(0)
    P, Q = _proj_call(n_pad, x_pad, p[pre + "_proj_W"],
                      p[pre + "_proj_b"].reshape(1, -1), wi, bi, wj)
    for l in range(N_LAYERS):
        parts = edge_pass(P, Q, ei4)
        wl = p[f"{pre}_l{l}_lin_W"]
        bl = p[f"{pre}_l{l}_lin_b"].reshape(1, -1)
        if l < N_LAYERS - 1:
            wi, bi, wj = msg_w(l + 1)
            P, Q = _mid_call(n_pad, parts, wl, bl, wi, bi, wj)
        else:
            return _pool_call(n_pad, n, parts, wl, bl)


def kernel(tree_x, tree_edge_index, graph_x, graph_edge_index, params):
    p = params
    tsum = _encode(tree_x, tree_edge_index, p, "tree", _TREE)
    gsum = _encode(graph_x, graph_edge_index, p, "graph", _GRAPH)
    eps = jax.random.normal(jax.random.key(42), (1, ZDIM), dtype=jnp.float32)
    logits, node_feats, mu, logvar = _head_call(tsum, gsum, eps, p)
    frags_logits = jnp.broadcast_to(logits[:, None, :], (1, MAX_TREE_NODES, VOCAB))
    return (frags_logits, node_feats, mu, logvar)
